# baseline (speedup 1.0000x reference)
import dataclasses
import functools

import jax
import jax.numpy as jnp
from jax import lax
from jax.experimental import pallas as pl
from jax.experimental.pallas import tpu as pltpu
from jax.experimental.pallas import tpu_sc as plsc

_F32 = jnp.float32
_NCORES = 2
_NSUB = 16
_NTILES = _NCORES * _NSUB


def _multi_mm_body(x_ref, *refs):
    nw = len(refs) // 2
    xb = x_ref[...]
    for w_ref, o_ref in zip(refs[:nw], refs[nw:]):
        o_ref[...] = jnp.dot(xb, w_ref[...], preferred_element_type=_F32)


def _multi_mm(x, ws, block_rows):
    n, d = x.shape
    grid = n // block_rows
    return pl.pallas_call(
        _multi_mm_body,
        grid=(grid,),
        in_specs=[pl.BlockSpec((block_rows, d), lambda i: (i, 0))]
        + [pl.BlockSpec((d, w.shape[1]), lambda i: (0, 0)) for w in ws],
        out_specs=[pl.BlockSpec((block_rows, w.shape[1]), lambda i: (i, 0))
                   for w in ws],
        out_shape=[jax.ShapeDtypeStruct((n, w.shape[1]), _F32) for w in ws],
    )(x, *ws)


def _finish1_body(acc0_ref, acc1_ref, den0_ref, den1_ref, b_ref, g_ref,
                  be_ref, wl_ref, wr_ref, xl_ref, xr_ref):
    acc = jnp.concatenate(
        [acc0_ref[0] + acc0_ref[1], acc1_ref[0] + acc1_ref[1]], axis=1)
    den0 = den0_ref[0] + den0_ref[1]
    den1 = den1_ref[0] + den1_ref[1]
    r = acc.shape[0]
    denf = jnp.concatenate(
        [jnp.broadcast_to(den0[:, 0:1], (r, 64)),
         jnp.broadcast_to(den1[:, 0:1], (r, 64))], axis=1)
    h = acc / (denf + 1e-16) + b_ref[...]
    mu = jnp.mean(h, axis=1, keepdims=True)
    var = jnp.mean((h - mu) ** 2, axis=1, keepdims=True)
    hn = (h - mu) * lax.rsqrt(var + 1e-5) * g_ref[...] + be_ref[...]
    h = jnp.maximum(hn, 0.0)
    xl_ref[...] = jnp.dot(h, wl_ref[...], preferred_element_type=_F32)
    xr_ref[...] = jnp.dot(h, wr_ref[...], preferred_element_type=_F32)


def _finish1(acc0, acc1, den0, den1, b, g, be, wl, wr, block_rows):
    n = acc0.shape[1]
    dh = acc0.shape[2]
    k = wl.shape[1]
    grid = n // block_rows
    return pl.pallas_call(
        _finish1_body,
        grid=(grid,),
        in_specs=[
            pl.BlockSpec((2, block_rows, dh), lambda i: (0, i, 0)),
            pl.BlockSpec((2, block_rows, dh), lambda i: (0, i, 0)),
            pl.BlockSpec((2, block_rows, 16), lambda i: (0, i, 0)),
            pl.BlockSpec((2, block_rows, 16), lambda i: (0, i, 0)),
            pl.BlockSpec((1, 2 * dh), lambda i: (0, 0)),
            pl.BlockSpec((1, 2 * dh), lambda i: (0, 0)),
            pl.BlockSpec((1, 2 * dh), lambda i: (0, 0)),
            pl.BlockSpec((2 * dh, k), lambda i: (0, 0)),
            pl.BlockSpec((2 * dh, k), lambda i: (0, 0)),
        ],
        out_specs=[
            pl.BlockSpec((block_rows, k), lambda i: (i, 0)),
            pl.BlockSpec((block_rows, k), lambda i: (i, 0)),
        ],
        out_shape=[
            jax.ShapeDtypeStruct((n, k), _F32),
            jax.ShapeDtypeStruct((n, k), _F32),
        ],
    )(acc0, acc1, den0, den1, b, g, be, wl, wr)


def _finish2_body(accp_ref, denp_ref, b_ref, g_ref, be_ref, out_ref):
    acc = accp_ref[0] + accp_ref[1]
    den = denp_ref[0] + denp_ref[1]
    r = acc.shape[0]
    denf = jnp.broadcast_to(den[:, 0:1], (r, 64))
    h = acc / (denf + 1e-16) + b_ref[...]
    mu = jnp.mean(h, axis=1, keepdims=True)
    var = jnp.mean((h - mu) ** 2, axis=1, keepdims=True)
    hn = (h - mu) * lax.rsqrt(var + 1e-5) * g_ref[...] + be_ref[...]
    out_ref[...] = jnp.maximum(hn, 0.0)


def _finish2(accp, denp, b, g, be, block_rows):
    n = accp.shape[1]
    dh = accp.shape[2]
    grid = n // block_rows
    return pl.pallas_call(
        _finish2_body,
        grid=(grid,),
        in_specs=[
            pl.BlockSpec((2, block_rows, dh), lambda i: (0, i, 0)),
            pl.BlockSpec((2, block_rows, 16), lambda i: (0, i, 0)),
            pl.BlockSpec((1, dh), lambda i: (0, 0)),
            pl.BlockSpec((1, dh), lambda i: (0, 0)),
            pl.BlockSpec((1, dh), lambda i: (0, 0)),
        ],
        out_specs=pl.BlockSpec((block_rows, dh), lambda i: (i, 0)),
        out_shape=jax.ShapeDtypeStruct((n, dh), _F32),
    )(accp, denp, b, g, be)


def _make_edge_pass(n, e, dh, heads, chunk):
    assert e % _NTILES == 0
    ep = e // _NTILES
    assert ep % chunk == 0 and chunk % 16 == 0 and chunk <= 128
    ncht = ep // chunk
    assert ncht >= 4
    rows_pt = (n // _NSUB) & ~7
    rows_tail = n - rows_pt * _NSUB
    assert rows_tail % 8 == 0
    nk = dh // 16
    kh = nk // heads
    mesh = plsc.VectorSubcoreMesh(core_axis_name="c", subcore_axis_name="s")
    cp = pltpu.CompilerParams()
    if "needs_layout_passes" in pltpu.CompilerParams.__dataclass_fields__:
        cp = dataclasses.replace(cp, needs_layout_passes=False)
    if "use_tc_tiling_on_sc" in pltpu.CompilerParams.__dataclass_fields__:
        cp = dataclasses.replace(cp, use_tc_tiling_on_sc=False)

    @functools.partial(
        pl.kernel,
        compiler_params=cp,
        out_type=[
            jax.ShapeDtypeStruct((_NCORES, n, dh), _F32),
            jax.ShapeDtypeStruct((_NCORES, n, 16), _F32),
        ],
        mesh=mesh,
        scratch_types=(
            [pltpu.VMEM((ep,), jnp.int32)] * 2
            + [pltpu.VMEM((chunk,), jnp.int32)] * 3
            + [pltpu.VMEM((chunk, dh), _F32)] * 3
            + [pltpu.VMEM((chunk, dh), _F32)] * 3
            + [pltpu.VMEM((chunk, dh), _F32)] * 3
            + [pltpu.VMEM((chunk, 16), _F32)] * 3
            + [
                pltpu.VMEM((dh,), _F32),
                pltpu.VMEM((256,), _F32),
                pltpu.VMEM((256,), _F32),
                pltpu.VMEM_SHARED((n, dh), _F32),
                pltpu.VMEM_SHARED((n, 16), _F32),
            ]
            + [pltpu.SemaphoreType.DMA] * 6
        ),
    )
    def edge_kernel(src_hbm, dst_hbm, xl_hbm, xr_hbm, ea_hbm, att_hbm,
                    zacc_hbm, zden_hbm, acc_out, den_out,
                    srca_v, dsta_v, dv0, dv1, dv2, xl0, xl1, xl2,
                    xr0, xr1, xr2, ea0, ea1, ea2, dn0, dn1, dn2, att_v,
                    t0_v, t1_v, acc_sh, den_sh, is0, is1, is2,
                    os0, os1, os2):
        dvs = (dv0, dv1, dv2)
        xls = (xl0, xl1, xl2)
        xrs = (xr0, xr1, xr2)
        eas = (ea0, ea1, ea2)
        dns = (dn0, dn1, dn2)
        isem = (is0, is1, is2)
        osem = (os0, os1, os2)
        c = lax.axis_index("c")
        s = lax.axis_index("s")
        wid = c * _NSUB + s
        r0 = s * rows_pt

        pltpu.sync_copy(zacc_hbm.at[pl.ds(r0, rows_pt)],
                        acc_sh.at[pl.ds(r0, rows_pt)])
        pltpu.sync_copy(zden_hbm.at[pl.ds(r0, rows_pt)],
                        den_sh.at[pl.ds(r0, rows_pt)])
        if rows_tail:
            tail0 = rows_pt * _NSUB

            @pl.when(s == 0)
            def _():
                pltpu.sync_copy(zacc_hbm.at[pl.ds(tail0, rows_tail)],
                                acc_sh.at[pl.ds(tail0, rows_tail)])
                pltpu.sync_copy(zden_hbm.at[pl.ds(tail0, rows_tail)],
                                den_sh.at[pl.ds(tail0, rows_tail)])
        pltpu.sync_copy(att_hbm, att_v)
        base0 = wid * ep
        pltpu.sync_copy(src_hbm.at[pl.ds(base0, ep)], srca_v)
        pltpu.sync_copy(dst_hbm.at[pl.ds(base0, ep)], dsta_v)

        zero16 = jnp.zeros((16,), _F32)
        for b in range(3):
            den_b = dns[b]

            @pl.loop(0, chunk)
            def _(rr):
                den_b[rr, pl.ds(0, 16)] = zero16

        plsc.subcore_barrier()

        iota = lax.iota(jnp.int32, 16)

        def issue_in(item, b):
            base = base0 + item * chunk
            pltpu.async_copy(dst_hbm.at[pl.ds(base, chunk)], dvs[b], isem[b])
            pltpu.async_copy(
                xl_hbm.at[srca_v.at[pl.ds(item * chunk, chunk)]],
                xls[b], isem[b])
            pltpu.async_copy(
                xr_hbm.at[dsta_v.at[pl.ds(item * chunk, chunk)]],
                xrs[b], isem[b])
            pltpu.async_copy(ea_hbm.at[pl.ds(base, chunk)], eas[b], isem[b])

        def wait_in(b):
            pltpu.make_async_copy(dst_hbm.at[pl.ds(0, chunk)], dvs[b],
                                  isem[b]).wait()
            pltpu.make_async_copy(zacc_hbm.at[pl.ds(0, chunk)], xls[b],
                                  isem[b]).wait()
            pltpu.make_async_copy(zacc_hbm.at[pl.ds(0, chunk)], xrs[b],
                                  isem[b]).wait()
            pltpu.make_async_copy(zacc_hbm.at[pl.ds(0, chunk)], eas[b],
                                  isem[b]).wait()

        def issue_out(b):
            pltpu.async_copy(xrs[b], acc_sh.at[dvs[b]], osem[b], add=True)
            pltpu.async_copy(dns[b], den_sh.at[dvs[b]], osem[b], add=True)

        def wait_out(b):
            pltpu.make_async_copy(zacc_hbm.at[pl.ds(0, chunk)], xrs[b],
                                  osem[b]).wait()
            pltpu.make_async_copy(zden_hbm.at[pl.ds(0, chunk)], dns[b],
                                  osem[b]).wait()

        def compute(b):
            xl_v, xr_v, ea_v, den_v = xls[b], xrs[b], eas[b], dns[b]
            att_regs = [att_v[pl.ds(k * 16, 16)] for k in range(nk)]

            @pl.loop(0, chunk, step=16)
            def _(g0):
                for ei in range(16):
                    row = g0 + ei
                    for h in range(heads):
                        acc = None
                        for kk in range(kh):
                            k = h * kh + kk
                            m = (xl_v[row, pl.ds(k * 16, 16)]
                                 + xr_v[row, pl.ds(k * 16, 16)]
                                 + ea_v[row, pl.ds(k * 16, 16)])
                            lr = jnp.maximum(m, 0.0) + 0.2 * jnp.minimum(m, 0.0)
                            w = lr * att_regs[k]
                            acc = w if acc is None else acc + w
                        tbuf = t0_v if h == 0 else t1_v
                        tbuf[pl.ds(ei * 16, 16)] = acc
                rows_idx = g0 + iota
                for h in range(heads):
                    tbuf = t0_v if h == 0 else t1_v
                    lg = None
                    for l in range(16):
                        v = plsc.load_gather(tbuf, [iota * 16 + l])
                        lg = v if lg is None else lg + v
                    ex = jnp.exp(lg)
                    plsc.store_scatter(
                        den_v, [rows_idx, jnp.full((16,), h, jnp.int32)], ex)
                    for ei in range(16):
                        row = g0 + ei
                        sc = lax.gather(
                            ex, jnp.full((16, 1), ei, jnp.int32),
                            lax.GatherDimensionNumbers(
                                offset_dims=(), collapsed_slice_dims=(0,),
                                start_index_map=(0,)),
                            slice_sizes=(1,),
                            mode=lax.GatherScatterMode.PROMISE_IN_BOUNDS)
                        for kk in range(kh):
                            k = h * kh + kk
                            xr_v[row, pl.ds(k * 16, 16)] = (
                                xl_v[row, pl.ds(k * 16, 16)] * sc)

        issue_in(0, 0)
        nsteps = -(-ncht // 3)

        @pl.loop(0, nsteps)
        def _(p):
            for bi in range(3):
                item = p * 3 + bi

                @pl.when(item < ncht)
                def _(item=item, bi=bi):
                    nxt = item + 1
                    bn = (bi + 1) % 3

                    @pl.when(jnp.logical_and(nxt < ncht, item >= 2))
                    def _():
                        wait_out(bn)

                    @pl.when(nxt < ncht)
                    def _():
                        issue_in(nxt, bn)

                    wait_in(bi)
                    compute(bi)
                    issue_out(bi)

        for last in range(ncht - 3, ncht):
            wait_out(last % 3)

        plsc.subcore_barrier()
        pltpu.sync_copy(acc_sh.at[pl.ds(r0, rows_pt)],
                        acc_out.at[c, pl.ds(r0, rows_pt)])
        pltpu.sync_copy(den_sh.at[pl.ds(r0, rows_pt)],
                        den_out.at[c, pl.ds(r0, rows_pt)])
        if rows_tail:
            tail0 = rows_pt * _NSUB

            @pl.when(s == 0)
            def _():
                pltpu.sync_copy(acc_sh.at[pl.ds(tail0, rows_tail)],
                                acc_out.at[c, pl.ds(tail0, rows_tail)])
                pltpu.sync_copy(den_sh.at[pl.ds(tail0, rows_tail)],
                                den_out.at[c, pl.ds(tail0, rows_tail)])

    return edge_kernel


def kernel(x, edge_index, edge_attr, Wl1, Wr1, We1, att1, b1, g1, be1,
           Wl2, Wr2, We2, att2, b2, g2, be2):
    n, d = x.shape
    e = edge_index.shape[1]
    d2 = Wl2.shape[1]
    heads = att1.shape[0]
    dh = Wl1.shape[1] // heads

    src = edge_index[0]
    dst = edge_index[1]

    wl_heads = [Wl1[:, h * dh:(h + 1) * dh] for h in range(heads)]
    wr_heads = [Wr1[:, h * dh:(h + 1) * dh] for h in range(heads)]
    xlr = _multi_mm(x, wl_heads + wr_heads, block_rows=400)
    xl_h, xr_h = xlr[:heads], xlr[heads:]
    we_heads = [We1[:, h * dh:(h + 1) * dh] for h in range(heads)]
    eas = _multi_mm(edge_attr, we_heads + [We2], block_rows=3200)
    ea_h, ea2 = eas[:heads], eas[heads]

    z = jnp.zeros((n, dh), _F32)
    zd = jnp.zeros((n, 16), _F32)

    edge = _make_edge_pass(n, e, dh, 1, chunk=80)
    accd = [edge(src, dst, xl_h[h], xr_h[h], ea_h[h], att1[h], z, zd)
            for h in range(heads)]

    xl2, xr2 = _finish1(accd[0][0], accd[1][0], accd[0][1], accd[1][1],
                        b1.reshape(1, -1), g1.reshape(1, -1),
                        be1.reshape(1, -1), Wl2, Wr2, block_rows=400)

    acc2, den2 = edge(src, dst, xl2, xr2, ea2, att2.reshape(-1), z, zd)

    return _finish2(acc2, den2, b2.reshape(1, -1), g2.reshape(1, -1),
                    be2.reshape(1, -1), block_rows=400)


# speedup vs baseline: 1.2770x; 1.2770x over previous
import dataclasses
import functools

import jax
import jax.numpy as jnp
from jax import lax
from jax.experimental import pallas as pl
from jax.experimental.pallas import tpu as pltpu
from jax.experimental.pallas import tpu_sc as plsc

_F32 = jnp.float32
_NCORES = 2
_NSUB = 16
_NTILES = _NCORES * _NSUB


def _multi_mm_body(x_ref, *refs):
    nw = len(refs) // 2
    xb = x_ref[...]
    for w_ref, o_ref in zip(refs[:nw], refs[nw:]):
        o_ref[...] = jnp.dot(xb, w_ref[...], preferred_element_type=_F32)


def _multi_mm(x, ws, block_rows):
    n, d = x.shape
    grid = n // block_rows
    return pl.pallas_call(
        _multi_mm_body,
        grid=(grid,),
        in_specs=[pl.BlockSpec((block_rows, d), lambda i: (i, 0))]
        + [pl.BlockSpec((d, w.shape[1]), lambda i: (0, 0)) for w in ws],
        out_specs=[pl.BlockSpec((block_rows, w.shape[1]), lambda i: (i, 0))
                   for w in ws],
        out_shape=[jax.ShapeDtypeStruct((n, w.shape[1]), _F32) for w in ws],
    )(x, *ws)


def _finish1_body(acc0_ref, acc1_ref, den0_ref, den1_ref, b_ref, g_ref,
                  be_ref, wl_ref, wr_ref, xl_ref, xr_ref):
    acc = jnp.concatenate(
        [acc0_ref[0] + acc0_ref[1], acc1_ref[0] + acc1_ref[1]], axis=1)
    den0 = den0_ref[0] + den0_ref[1]
    den1 = den1_ref[0] + den1_ref[1]
    r = acc.shape[0]
    denf = jnp.concatenate(
        [jnp.broadcast_to(den0[:, 0:1], (r, 64)),
         jnp.broadcast_to(den1[:, 0:1], (r, 64))], axis=1)
    h = acc / (denf + 1e-16) + b_ref[...]
    mu = jnp.mean(h, axis=1, keepdims=True)
    var = jnp.mean((h - mu) ** 2, axis=1, keepdims=True)
    hn = (h - mu) * lax.rsqrt(var + 1e-5) * g_ref[...] + be_ref[...]
    h = jnp.maximum(hn, 0.0)
    xl_ref[...] = jnp.dot(h, wl_ref[...], preferred_element_type=_F32)
    xr_ref[...] = jnp.dot(h, wr_ref[...], preferred_element_type=_F32)


def _finish1(acc0, acc1, den0, den1, b, g, be, wl, wr, block_rows):
    n = acc0.shape[1]
    dh = acc0.shape[2]
    k = wl.shape[1]
    grid = n // block_rows
    return pl.pallas_call(
        _finish1_body,
        grid=(grid,),
        in_specs=[
            pl.BlockSpec((2, block_rows, dh), lambda i: (0, i, 0)),
            pl.BlockSpec((2, block_rows, dh), lambda i: (0, i, 0)),
            pl.BlockSpec((2, block_rows, 16), lambda i: (0, i, 0)),
            pl.BlockSpec((2, block_rows, 16), lambda i: (0, i, 0)),
            pl.BlockSpec((1, 2 * dh), lambda i: (0, 0)),
            pl.BlockSpec((1, 2 * dh), lambda i: (0, 0)),
            pl.BlockSpec((1, 2 * dh), lambda i: (0, 0)),
            pl.BlockSpec((2 * dh, k), lambda i: (0, 0)),
            pl.BlockSpec((2 * dh, k), lambda i: (0, 0)),
        ],
        out_specs=[
            pl.BlockSpec((block_rows, k), lambda i: (i, 0)),
            pl.BlockSpec((block_rows, k), lambda i: (i, 0)),
        ],
        out_shape=[
            jax.ShapeDtypeStruct((n, k), _F32),
            jax.ShapeDtypeStruct((n, k), _F32),
        ],
    )(acc0, acc1, den0, den1, b, g, be, wl, wr)


def _finish2_body(accp_ref, denp_ref, b_ref, g_ref, be_ref, out_ref):
    acc = accp_ref[0] + accp_ref[1]
    den = denp_ref[0] + denp_ref[1]
    r = acc.shape[0]
    denf = jnp.broadcast_to(den[:, 0:1], (r, 64))
    h = acc / (denf + 1e-16) + b_ref[...]
    mu = jnp.mean(h, axis=1, keepdims=True)
    var = jnp.mean((h - mu) ** 2, axis=1, keepdims=True)
    hn = (h - mu) * lax.rsqrt(var + 1e-5) * g_ref[...] + be_ref[...]
    out_ref[...] = jnp.maximum(hn, 0.0)


def _finish2(accp, denp, b, g, be, block_rows):
    n = accp.shape[1]
    dh = accp.shape[2]
    grid = n // block_rows
    return pl.pallas_call(
        _finish2_body,
        grid=(grid,),
        in_specs=[
            pl.BlockSpec((2, block_rows, dh), lambda i: (0, i, 0)),
            pl.BlockSpec((2, block_rows, 16), lambda i: (0, i, 0)),
            pl.BlockSpec((1, dh), lambda i: (0, 0)),
            pl.BlockSpec((1, dh), lambda i: (0, 0)),
            pl.BlockSpec((1, dh), lambda i: (0, 0)),
        ],
        out_specs=pl.BlockSpec((block_rows, dh), lambda i: (i, 0)),
        out_shape=jax.ShapeDtypeStruct((n, dh), _F32),
    )(accp, denp, b, g, be)


def _make_edge_pass(n, e, dh, heads, chunk):
    assert e % _NTILES == 0
    ep = e // _NTILES
    assert ep % chunk == 0 and chunk % 16 == 0 and chunk <= 128
    ncht = ep // chunk
    assert ncht >= 4
    rows_pt = (n // _NSUB) & ~7
    rows_tail = n - rows_pt * _NSUB
    assert rows_tail % 8 == 0
    nk = dh // 16
    kh = nk // heads
    mesh = plsc.VectorSubcoreMesh(core_axis_name="c", subcore_axis_name="s")
    cp = pltpu.CompilerParams()
    if "needs_layout_passes" in pltpu.CompilerParams.__dataclass_fields__:
        cp = dataclasses.replace(cp, needs_layout_passes=False)
    if "use_tc_tiling_on_sc" in pltpu.CompilerParams.__dataclass_fields__:
        cp = dataclasses.replace(cp, use_tc_tiling_on_sc=False)

    @functools.partial(
        pl.kernel,
        compiler_params=cp,
        out_type=[
            jax.ShapeDtypeStruct((_NCORES, n, dh), _F32),
            jax.ShapeDtypeStruct((_NCORES, n, 16), _F32),
        ],
        mesh=mesh,
        scratch_types=(
            [pltpu.VMEM((ep,), jnp.int32)] * 2
            + [pltpu.VMEM((chunk,), jnp.int32)] * 3
            + [pltpu.VMEM((chunk, dh), _F32)] * 3
            + [pltpu.VMEM((chunk, dh), _F32)] * 3
            + [pltpu.VMEM((chunk, dh), _F32)] * 3
            + [pltpu.VMEM((chunk, 16), _F32)] * 3
            + [
                pltpu.VMEM((dh,), _F32),
                pltpu.VMEM((256,), _F32),
                pltpu.VMEM((256,), _F32),
                pltpu.VMEM_SHARED((n, dh), _F32),
                pltpu.VMEM_SHARED((n, 16), _F32),
            ]
            + [pltpu.SemaphoreType.DMA] * 6
        ),
    )
    def edge_kernel(src_hbm, dst_hbm, xl_hbm, xr_hbm, ea_hbm, att_hbm,
                    zacc_hbm, zden_hbm, acc_out, den_out,
                    srca_v, dsta_v, dv0, dv1, dv2, xl0, xl1, xl2,
                    xr0, xr1, xr2, ea0, ea1, ea2, dn0, dn1, dn2, att_v,
                    t0_v, t1_v, acc_sh, den_sh, is0, is1, is2,
                    os0, os1, os2):
        dvs = (dv0, dv1, dv2)
        xls = (xl0, xl1, xl2)
        xrs = (xr0, xr1, xr2)
        eas = (ea0, ea1, ea2)
        dns = (dn0, dn1, dn2)
        isem = (is0, is1, is2)
        osem = (os0, os1, os2)
        c = lax.axis_index("c")
        s = lax.axis_index("s")
        wid = c * _NSUB + s
        r0 = s * rows_pt

        pltpu.sync_copy(zacc_hbm.at[pl.ds(r0, rows_pt)],
                        acc_sh.at[pl.ds(r0, rows_pt)])
        pltpu.sync_copy(zden_hbm.at[pl.ds(r0, rows_pt)],
                        den_sh.at[pl.ds(r0, rows_pt)])
        if rows_tail:
            tail0 = rows_pt * _NSUB

            @pl.when(s == 0)
            def _():
                pltpu.sync_copy(zacc_hbm.at[pl.ds(tail0, rows_tail)],
                                acc_sh.at[pl.ds(tail0, rows_tail)])
                pltpu.sync_copy(zden_hbm.at[pl.ds(tail0, rows_tail)],
                                den_sh.at[pl.ds(tail0, rows_tail)])
        pltpu.sync_copy(att_hbm, att_v)
        base0 = wid * ep
        pltpu.sync_copy(src_hbm.at[pl.ds(base0, ep)], srca_v)
        pltpu.sync_copy(dst_hbm.at[pl.ds(base0, ep)], dsta_v)

        zero16 = jnp.zeros((16,), _F32)
        for b in range(3):
            den_b = dns[b]

            @pl.loop(0, chunk)
            def _(rr):
                den_b[rr, pl.ds(0, 16)] = zero16

        plsc.subcore_barrier()

        iota = lax.iota(jnp.int32, 16)

        def issue_in(item, b):
            base = base0 + item * chunk
            pltpu.async_copy(dst_hbm.at[pl.ds(base, chunk)], dvs[b], isem[b])
            pltpu.async_copy(
                xl_hbm.at[srca_v.at[pl.ds(item * chunk, chunk)]],
                xls[b], isem[b])
            pltpu.async_copy(
                xr_hbm.at[dsta_v.at[pl.ds(item * chunk, chunk)]],
                xrs[b], isem[b])
            pltpu.async_copy(ea_hbm.at[pl.ds(base, chunk)], eas[b], isem[b])

        def wait_in(b):
            pltpu.make_async_copy(dst_hbm.at[pl.ds(0, chunk)], dvs[b],
                                  isem[b]).wait()
            pltpu.make_async_copy(zacc_hbm.at[pl.ds(0, chunk)], xls[b],
                                  isem[b]).wait()
            pltpu.make_async_copy(zacc_hbm.at[pl.ds(0, chunk)], xrs[b],
                                  isem[b]).wait()
            pltpu.make_async_copy(zacc_hbm.at[pl.ds(0, chunk)], eas[b],
                                  isem[b]).wait()

        def issue_out(b):
            pltpu.async_copy(xrs[b], acc_sh.at[dvs[b]], osem[b], add=True)
            pltpu.async_copy(dns[b], den_sh.at[dvs[b]], osem[b], add=True)

        def wait_out(b):
            pltpu.make_async_copy(zacc_hbm.at[pl.ds(0, chunk)], xrs[b],
                                  osem[b]).wait()
            pltpu.make_async_copy(zden_hbm.at[pl.ds(0, chunk)], dns[b],
                                  osem[b]).wait()

        gdn = lax.GatherDimensionNumbers(
            offset_dims=(), collapsed_slice_dims=(0,), start_index_map=(0,))

        def vperm(v, idx16):
            return lax.gather(v, idx16.reshape(16, 1), gdn, slice_sizes=(1,),
                              mode=lax.GatherScatterMode.PROMISE_IN_BOUNDS)

        xor_idx = {k: iota ^ k for k in (8, 4, 2, 1)}
        keep_lo = {k: (iota & k) == 0 for k in (8, 4, 2, 1)}

        def lane_sums(vecs):
            for k in (8, 4, 2, 1):
                n2 = len(vecs) // 2
                nxt = []
                for i in range(n2):
                    u = vecs[i] + vperm(vecs[i], xor_idx[k])
                    w = vecs[i + n2] + vperm(vecs[i + n2], xor_idx[k])
                    nxt.append(jnp.where(keep_lo[k], u, vperm(w, xor_idx[k])))
                vecs = nxt
            return vecs[0]

        def compute(b):
            xl_v, xr_v, ea_v, den_v = xls[b], xrs[b], eas[b], dns[b]
            att_regs = [att_v[pl.ds(k * 16, 16)] for k in range(nk)]

            @pl.loop(0, chunk, step=16)
            def _(g0):
                rows_idx = g0 + iota
                for h in range(heads):
                    accs = []
                    for ei in range(16):
                        row = g0 + ei
                        acc = None
                        for kk in range(kh):
                            k = h * kh + kk
                            m = (xl_v[row, pl.ds(k * 16, 16)]
                                 + xr_v[row, pl.ds(k * 16, 16)]
                                 + ea_v[row, pl.ds(k * 16, 16)])
                            lr = jnp.maximum(m, 0.0) + 0.2 * jnp.minimum(m, 0.0)
                            w = lr * att_regs[k]
                            acc = w if acc is None else acc + w
                        accs.append(acc)
                    ex = jnp.exp(lane_sums(accs))
                    plsc.store_scatter(
                        den_v, [rows_idx, jnp.full((16,), h, jnp.int32)], ex)
                    for ei in range(16):
                        row = g0 + ei
                        sc = vperm(ex, jnp.full((16,), ei, jnp.int32))
                        for kk in range(kh):
                            k = h * kh + kk
                            xr_v[row, pl.ds(k * 16, 16)] = (
                                xl_v[row, pl.ds(k * 16, 16)] * sc)

        issue_in(0, 0)
        nsteps = -(-ncht // 3)

        @pl.loop(0, nsteps)
        def _(p):
            for bi in range(3):
                item = p * 3 + bi

                @pl.when(item < ncht)
                def _(item=item, bi=bi):
                    nxt = item + 1
                    bn = (bi + 1) % 3

                    @pl.when(jnp.logical_and(nxt < ncht, item >= 2))
                    def _():
                        wait_out(bn)

                    @pl.when(nxt < ncht)
                    def _():
                        issue_in(nxt, bn)

                    wait_in(bi)
                    compute(bi)
                    issue_out(bi)

        for last in range(ncht - 3, ncht):
            wait_out(last % 3)

        plsc.subcore_barrier()
        pltpu.sync_copy(acc_sh.at[pl.ds(r0, rows_pt)],
                        acc_out.at[c, pl.ds(r0, rows_pt)])
        pltpu.sync_copy(den_sh.at[pl.ds(r0, rows_pt)],
                        den_out.at[c, pl.ds(r0, rows_pt)])
        if rows_tail:
            tail0 = rows_pt * _NSUB

            @pl.when(s == 0)
            def _():
                pltpu.sync_copy(acc_sh.at[pl.ds(tail0, rows_tail)],
                                acc_out.at[c, pl.ds(tail0, rows_tail)])
                pltpu.sync_copy(den_sh.at[pl.ds(tail0, rows_tail)],
                                den_out.at[c, pl.ds(tail0, rows_tail)])

    return edge_kernel


def kernel(x, edge_index, edge_attr, Wl1, Wr1, We1, att1, b1, g1, be1,
           Wl2, Wr2, We2, att2, b2, g2, be2):
    n, d = x.shape
    e = edge_index.shape[1]
    d2 = Wl2.shape[1]
    heads = att1.shape[0]
    dh = Wl1.shape[1] // heads

    src = edge_index[0]
    dst = edge_index[1]

    wl_heads = [Wl1[:, h * dh:(h + 1) * dh] for h in range(heads)]
    wr_heads = [Wr1[:, h * dh:(h + 1) * dh] for h in range(heads)]
    xlr = _multi_mm(x, wl_heads + wr_heads, block_rows=400)
    xl_h, xr_h = xlr[:heads], xlr[heads:]
    we_heads = [We1[:, h * dh:(h + 1) * dh] for h in range(heads)]
    eas = _multi_mm(edge_attr, we_heads + [We2], block_rows=3200)
    ea_h, ea2 = eas[:heads], eas[heads]

    z = jnp.zeros((n, dh), _F32)
    zd = jnp.zeros((n, 16), _F32)

    edge = _make_edge_pass(n, e, dh, 1, chunk=80)
    accd = [edge(src, dst, xl_h[h], xr_h[h], ea_h[h], att1[h], z, zd)
            for h in range(heads)]

    xl2, xr2 = _finish1(accd[0][0], accd[1][0], accd[0][1], accd[1][1],
                        b1.reshape(1, -1), g1.reshape(1, -1),
                        be1.reshape(1, -1), Wl2, Wr2, block_rows=400)

    acc2, den2 = edge(src, dst, xl2, xr2, ea2, att2.reshape(-1), z, zd)

    return _finish2(acc2, den2, b2.reshape(1, -1), g2.reshape(1, -1),
                    be2.reshape(1, -1), block_rows=400)
